# baseline (speedup 1.0000x reference)
import functools

import jax
import jax.numpy as jnp
from jax import lax
from jax.experimental import pallas as pl
from jax.experimental.pallas import tpu as pltpu
from jax.experimental.pallas import tpu_sc as plsc

NC = 2
NS = 16
L = 16
NW = NC * NS

Q = 1024
K = 100000

T_TC = 66560
SC_N = K - T_TC
CHUNK = ((SC_N + NW * L - 1) // (NW * L)) * L
SC_KPAD = CHUNK * NW
KG = T_TC + SC_KPAD

QPW = Q // NW
UNROLL = 16
G = 2
KUNROLL = 8
TKB = T_TC // 128
IMAX = 0x7FFFFFFF

_mesh = functools.partial(
    plsc.VectorSubcoreMesh, core_axis_name="c", subcore_axis_name="s")


def _tc_body(qx_ref, qy_ref, mx_ref, my_ref, outd_ref, outi_ref):
    lane_iota = lax.broadcasted_iota(jnp.int32, (8, 128), 1)
    inf8 = jnp.full((8, 128), jnp.inf, jnp.float32)
    zero8 = jnp.zeros((8, 128), jnp.int32)

    def per_qblock(qb, _):
        qxb = jnp.broadcast_to(qx_ref[pl.ds(qb * 8, 8), :], (8, 128))
        qyb = jnp.broadcast_to(qy_ref[pl.ds(qb * 8, 8), :], (8, 128))

        def inner(t, carry):
            bds, bis = carry
            nbds, nbis = [], []
            for u in range(KUNROLL):
                kb = t * KUNROLL + u
                mxb = jnp.broadcast_to(mx_ref[pl.ds(kb, 1), :], (8, 128))
                myb = jnp.broadcast_to(my_ref[pl.ds(kb, 1), :], (8, 128))
                dx = qxb - mxb
                dy = qyb - myb
                d = dx * dx + dy * dy
                lt = d < bds[u]
                nbds.append(jnp.where(lt, d, bds[u]))
                nbis.append(jnp.where(lt, jnp.full((8, 128), kb, jnp.int32),
                                      bis[u]))
            return tuple(nbds), tuple(nbis)

        bds, bis = lax.fori_loop(
            0, TKB // KUNROLL, inner,
            ((inf8,) * KUNROLL, (zero8,) * KUNROLL))

        bd = bds[0]
        bi = bis[0] * 128 + lane_iota
        for u in range(1, KUNROLL):
            du = bds[u]
            iu = bis[u] * 128 + lane_iota
            lt = (du < bd) | ((du == bd) & (iu < bi))
            bd = jnp.where(lt, du, bd)
            bi = jnp.where(lt, iu, bi)

        outd_ref[qb] = bd
        outi_ref[qb] = bi
        return 0

    lax.fori_loop(0, Q // 8, per_qblock, 0)


_tc_stage = pl.pallas_call(
    _tc_body,
    out_shape=(
        jax.ShapeDtypeStruct((Q // 8, 8, 128), jnp.float32),
        jax.ShapeDtypeStruct((Q // 8, 8, 128), jnp.int32),
    ),
)


@functools.partial(
    pl.kernel,
    out_type=(
        jax.ShapeDtypeStruct((NW, Q), jnp.float32),
        jax.ShapeDtypeStruct((NW, Q), jnp.int32),
    ),
    mesh=_mesh(),
    scratch_types=[
        pltpu.VMEM((CHUNK,), jnp.float32),
        pltpu.VMEM((CHUNK,), jnp.float32),
        pltpu.VMEM((Q,), jnp.float32),
        pltpu.VMEM((Q,), jnp.float32),
        pltpu.VMEM((Q,), jnp.float32),
        pltpu.VMEM((Q,), jnp.int32),
    ],
)
def _stage1_sc(qx_hbm, qy_hbm, mx_hbm, my_hbm, outd_hbm, outi_hbm,
               mxv, myv, qxv, qyv, bdv, biv):
    c = lax.axis_index("c")
    s = lax.axis_index("s")
    w = s * NC + c
    base = w * CHUNK

    pltpu.sync_copy(mx_hbm.at[pl.ds(base, CHUNK)], mxv)
    pltpu.sync_copy(my_hbm.at[pl.ds(base, CHUNK)], myv)
    pltpu.sync_copy(qx_hbm, qxv)
    pltpu.sync_copy(qy_hbm, qyv)

    def per_group(g, _):
        qxb = [qxv[pl.ds((g * G + j) * L, L)] for j in range(G)]
        qyb = [qyv[pl.ds((g * G + j) * L, L)] for j in range(G)]

        def inner(i, carry):
            bd, bi = carry
            k0 = i * UNROLL
            mxvec = mxv[pl.ds(k0, UNROLL)]
            myvec = myv[pl.ds(k0, UNROLL)]
            bd, bi = list(bd), list(bi)
            for u in range(UNROLL):
                k = k0 + u
                mxb = jnp.full((L,), mxvec[u], jnp.float32)
                myb = jnp.full((L,), myvec[u], jnp.float32)
                for j in range(G):
                    dx = qxb[j] - mxb
                    dy = qyb[j] - myb
                    d = dx * dx + dy * dy
                    lt = d < bd[j]
                    bd[j] = jnp.where(lt, d, bd[j])
                    bi[j] = jnp.where(lt, T_TC + base + k, bi[j])
            return tuple(bd), tuple(bi)

        bd0 = (jnp.full((L,), jnp.inf, jnp.float32),) * G
        bi0 = (jnp.zeros((L,), jnp.int32),) * G
        bd, bi = lax.fori_loop(0, CHUNK // UNROLL, inner, (bd0, bi0))
        for j in range(G):
            bdv[pl.ds((g * G + j) * L, L)] = bd[j]
            biv[pl.ds((g * G + j) * L, L)] = bi[j]
        return 0

    lax.fori_loop(0, Q // (L * G), per_group, 0)

    pltpu.sync_copy(bdv, outd_hbm.at[w])
    pltpu.sync_copy(biv, outi_hbm.at[w])


@functools.partial(
    pl.kernel,
    out_type=(
        jax.ShapeDtypeStruct((Q,), jnp.float32),
        jax.ShapeDtypeStruct((Q,), jnp.float32),
        jax.ShapeDtypeStruct((Q,), jnp.float32),
    ),
    mesh=_mesh(),
    scratch_types=[
        pltpu.VMEM((NW, QPW), jnp.float32),
        pltpu.VMEM((NW, QPW), jnp.int32),
        pltpu.VMEM((QPW, 128), jnp.float32),
        pltpu.VMEM((QPW, 128), jnp.int32),
        pltpu.VMEM((QPW,), jnp.int32),
        pltpu.VMEM((QPW,), jnp.float32),
        pltpu.VMEM((QPW,), jnp.float32),
        pltpu.VMEM((QPW,), jnp.float32),
        pltpu.SemaphoreType.DMA,
    ],
)
def _stage2(tcd_hbm, tci_hbm, dall_hbm, iall_hbm, mx_hbm, my_hbm, ts_hbm,
            px_hbm, py_hbm, lin_hbm,
            dbuf, ibuf, tdbuf, tibuf, biv, pxv, pyv, linv, sem):
    c = lax.axis_index("c")
    s = lax.axis_index("s")
    w = s * NC + c
    qbase = w * QPW

    copies = [
        pltpu.async_copy(tcd_hbm.at[pl.ds(qbase, QPW)], tdbuf, sem),
        pltpu.async_copy(tci_hbm.at[pl.ds(qbase, QPW)], tibuf, sem),
    ]
    for r in range(NW):
        copies.append(
            pltpu.async_copy(dall_hbm.at[r, pl.ds(qbase, QPW)], dbuf.at[r], sem))
        copies.append(
            pltpu.async_copy(iall_hbm.at[r, pl.ds(qbase, QPW)], ibuf.at[r], sem))
    for cp in copies:
        cp.wait()

    iota16 = lax.iota(jnp.int32, L)

    def merge_pair(a, b):
        da, ia = a
        db, ib = b
        take_b = (db < da) | ((db == da) & (ib < ia))
        return (jnp.where(take_b, db, da), jnp.where(take_b, ib, ia))

    for j in range(QPW // L):
        bd = jnp.full((L,), jnp.inf, jnp.float32)
        bi = jnp.zeros((L,), jnp.int32)
        for ql in range(L):
            q = j * L + ql
            cands = [(tdbuf[q, pl.ds(cc * L, L)], tibuf[q, pl.ds(cc * L, L)])
                     for cc in range(128 // L)]
            while len(cands) > 1:
                cands = [merge_pair(cands[i], cands[i + 1])
                         for i in range(0, len(cands), 2)]
            vd, vi = cands[0]
            best_d = vd[0]
            best_i = vi[0]
            for lidx in range(1, L):
                dl = vd[lidx]
                il = vi[lidx]
                t = (dl < best_d) | ((dl == best_d) & (il < best_i))
                best_d = jnp.where(t, dl, best_d)
                best_i = jnp.where(t, il, best_i)
            lane = iota16 == ql
            bd = jnp.where(lane, best_d, bd)
            bi = jnp.where(lane, best_i, bi)
        for r in range(NW):
            dr = dbuf[r, pl.ds(j * L, L)]
            ir = ibuf[r, pl.ds(j * L, L)]
            lt = dr < bd
            bd = jnp.where(lt, dr, bd)
            bi = jnp.where(lt, ir, bi)
        biv[pl.ds(j * L, L)] = bi

    pltpu.async_copy(mx_hbm.at[biv], pxv, sem).wait()
    pltpu.async_copy(my_hbm.at[biv], pyv, sem).wait()
    pltpu.async_copy(ts_hbm.at[biv], linv, sem).wait()

    pltpu.sync_copy(pxv, px_hbm.at[pl.ds(qbase, QPW)])
    pltpu.sync_copy(pyv, py_hbm.at[pl.ds(qbase, QPW)])
    pltpu.sync_copy(linv, lin_hbm.at[pl.ds(qbase, QPW)])


@jax.jit
def kernel(euclidean_data, maze_points, ts_proj):
    ed = euclidean_data.astype(maze_points.dtype)
    qx = ed[:, 0]
    qy = ed[:, 1]
    padv = jnp.full((KG - K,), 1e6, jnp.float32)
    mxg = jnp.concatenate([maze_points[:, 0], padv])
    myg = jnp.concatenate([maze_points[:, 1], padv])

    padtc = jnp.full((KUNROLL * 128,), 1e6, jnp.float32)
    tc_d2, tc_i2 = _tc_stage(
        qx[:, None], qy[:, None],
        jnp.concatenate([mxg[:T_TC], padtc]).reshape(TKB + KUNROLL, 128),
        jnp.concatenate([myg[:T_TC], padtc]).reshape(TKB + KUNROLL, 128))
    tc_d = tc_d2.reshape(Q, 128)
    tc_i = tc_i2.reshape(Q, 128)

    dall, iall = _stage1_sc(qx, qy, mxg[T_TC:], myg[T_TC:])
    px, py, lin = _stage2(tc_d, tc_i, dall, iall, mxg, myg, ts_proj)
    projected = jnp.stack([px, py], axis=-1)
    return projected, lin


# speedup vs baseline: 1.0753x; 1.0753x over previous
import functools

import jax
import jax.numpy as jnp
from jax import lax
from jax.experimental import pallas as pl
from jax.experimental.pallas import tpu as pltpu
from jax.experimental.pallas import tpu_sc as plsc

NC = 2
NS = 16
L = 16
NW = NC * NS

Q = 1024
K = 100000

T_TC = 66560
SC_N = K - T_TC
CHUNK = ((SC_N + NW * L - 1) // (NW * L)) * L
SC_KPAD = CHUNK * NW
KG = T_TC + SC_KPAD

QPW = Q // NW
UNROLL = 16
G = 1
KUNROLL = 8
TKB = T_TC // 128
IMAX = 0x7FFFFFFF

_mesh = functools.partial(
    plsc.VectorSubcoreMesh, core_axis_name="c", subcore_axis_name="s")


def _tc_body(qx_ref, qy_ref, mx_ref, my_ref, outd_ref, outi_ref):
    lane_iota = lax.broadcasted_iota(jnp.int32, (8, 128), 1)
    inf8 = jnp.full((8, 128), jnp.inf, jnp.float32)
    zero8 = jnp.zeros((8, 128), jnp.int32)

    def per_qblock(qb, _):
        qxb = jnp.broadcast_to(qx_ref[pl.ds(qb * 8, 8), :], (8, 128))
        qyb = jnp.broadcast_to(qy_ref[pl.ds(qb * 8, 8), :], (8, 128))

        def inner(t, carry):
            bds, bis = carry
            nbds, nbis = [], []
            for u in range(KUNROLL):
                kb = t * KUNROLL + u
                mxb = jnp.broadcast_to(mx_ref[pl.ds(kb, 1), :], (8, 128))
                myb = jnp.broadcast_to(my_ref[pl.ds(kb, 1), :], (8, 128))
                dx = qxb - mxb
                dy = qyb - myb
                d = dx * dx + dy * dy
                lt = d < bds[u]
                nbds.append(jnp.where(lt, d, bds[u]))
                nbis.append(jnp.where(lt, jnp.full((8, 128), kb, jnp.int32),
                                      bis[u]))
            return tuple(nbds), tuple(nbis)

        bds, bis = lax.fori_loop(
            0, TKB // KUNROLL, inner,
            ((inf8,) * KUNROLL, (zero8,) * KUNROLL))

        bd = bds[0]
        bi = bis[0] * 128 + lane_iota
        for u in range(1, KUNROLL):
            du = bds[u]
            iu = bis[u] * 128 + lane_iota
            lt = (du < bd) | ((du == bd) & (iu < bi))
            bd = jnp.where(lt, du, bd)
            bi = jnp.where(lt, iu, bi)

        outd_ref[qb] = bd
        outi_ref[qb] = bi
        return 0

    lax.fori_loop(0, Q // 8, per_qblock, 0)


_tc_stage = pl.pallas_call(
    _tc_body,
    out_shape=(
        jax.ShapeDtypeStruct((Q // 8, 8, 128), jnp.float32),
        jax.ShapeDtypeStruct((Q // 8, 8, 128), jnp.int32),
    ),
)


@functools.partial(
    pl.kernel,
    out_type=(
        jax.ShapeDtypeStruct((NW, Q), jnp.float32),
        jax.ShapeDtypeStruct((NW, Q), jnp.int32),
    ),
    mesh=_mesh(),
    scratch_types=[
        pltpu.VMEM((CHUNK,), jnp.float32),
        pltpu.VMEM((CHUNK,), jnp.float32),
        pltpu.VMEM((Q,), jnp.float32),
        pltpu.VMEM((Q,), jnp.float32),
        pltpu.VMEM((Q,), jnp.float32),
        pltpu.VMEM((Q,), jnp.int32),
    ],
)
def _stage1_sc(qx_hbm, qy_hbm, mx_hbm, my_hbm, outd_hbm, outi_hbm,
               mxv, myv, qxv, qyv, bdv, biv):
    c = lax.axis_index("c")
    s = lax.axis_index("s")
    w = s * NC + c
    base = jnp.minimum(w * CHUNK, SC_N - CHUNK)

    pltpu.sync_copy(mx_hbm.at[pl.ds(base, CHUNK)], mxv)
    pltpu.sync_copy(my_hbm.at[pl.ds(base, CHUNK)], myv)
    pltpu.sync_copy(qx_hbm, qxv)
    pltpu.sync_copy(qy_hbm, qyv)

    def per_group(g, _):
        qxb = [qxv[pl.ds((g * G + j) * L, L)] for j in range(G)]
        qyb = [qyv[pl.ds((g * G + j) * L, L)] for j in range(G)]

        def inner(i, carry):
            bd, bi = carry
            k0 = i * UNROLL
            mxvec = mxv[pl.ds(k0, UNROLL)]
            myvec = myv[pl.ds(k0, UNROLL)]
            bd, bi = list(bd), list(bi)
            for u in range(UNROLL):
                k = k0 + u
                mxb = jnp.full((L,), mxvec[u], jnp.float32)
                myb = jnp.full((L,), myvec[u], jnp.float32)
                for j in range(G):
                    dx = qxb[j] - mxb
                    dy = qyb[j] - myb
                    d = dx * dx + dy * dy
                    lt = d < bd[j]
                    bd[j] = jnp.where(lt, d, bd[j])
                    bi[j] = jnp.where(lt, T_TC + base + k, bi[j])
            return tuple(bd), tuple(bi)

        bd0 = (jnp.full((L,), jnp.inf, jnp.float32),) * G
        bi0 = (jnp.zeros((L,), jnp.int32),) * G
        bd, bi = lax.fori_loop(0, CHUNK // UNROLL, inner, (bd0, bi0))
        for j in range(G):
            bdv[pl.ds((g * G + j) * L, L)] = bd[j]
            biv[pl.ds((g * G + j) * L, L)] = bi[j]
        return 0

    lax.fori_loop(0, Q // (L * G), per_group, 0)

    pltpu.sync_copy(bdv, outd_hbm.at[w])
    pltpu.sync_copy(biv, outi_hbm.at[w])


@functools.partial(
    pl.kernel,
    out_type=(
        jax.ShapeDtypeStruct((Q,), jnp.float32),
        jax.ShapeDtypeStruct((Q,), jnp.float32),
        jax.ShapeDtypeStruct((Q,), jnp.float32),
    ),
    mesh=_mesh(),
    scratch_types=[
        pltpu.VMEM((NW, QPW), jnp.float32),
        pltpu.VMEM((NW, QPW), jnp.int32),
        pltpu.VMEM((QPW, 128), jnp.float32),
        pltpu.VMEM((QPW, 128), jnp.int32),
        pltpu.VMEM((QPW,), jnp.int32),
        pltpu.VMEM((QPW,), jnp.float32),
        pltpu.VMEM((QPW,), jnp.float32),
        pltpu.VMEM((QPW,), jnp.float32),
        pltpu.SemaphoreType.DMA,
    ],
)
def _stage2(tcd_hbm, tci_hbm, dall_hbm, iall_hbm, mx_hbm, my_hbm, ts_hbm,
            px_hbm, py_hbm, lin_hbm,
            dbuf, ibuf, tdbuf, tibuf, biv, pxv, pyv, linv, sem):
    c = lax.axis_index("c")
    s = lax.axis_index("s")
    w = s * NC + c
    qbase = w * QPW

    copies = [
        pltpu.async_copy(tcd_hbm.at[pl.ds(qbase, QPW)], tdbuf, sem),
        pltpu.async_copy(tci_hbm.at[pl.ds(qbase, QPW)], tibuf, sem),
    ]
    for r in range(NW):
        copies.append(
            pltpu.async_copy(dall_hbm.at[r, pl.ds(qbase, QPW)], dbuf.at[r], sem))
        copies.append(
            pltpu.async_copy(iall_hbm.at[r, pl.ds(qbase, QPW)], ibuf.at[r], sem))
    for cp in copies:
        cp.wait()

    iota16 = lax.iota(jnp.int32, L)

    def merge_pair(a, b):
        da, ia = a
        db, ib = b
        take_b = (db < da) | ((db == da) & (ib < ia))
        return (jnp.where(take_b, db, da), jnp.where(take_b, ib, ia))

    for j in range(QPW // L):
        bd = jnp.full((L,), jnp.inf, jnp.float32)
        bi = jnp.zeros((L,), jnp.int32)
        for ql in range(L):
            q = j * L + ql
            cands = [(tdbuf[q, pl.ds(cc * L, L)], tibuf[q, pl.ds(cc * L, L)])
                     for cc in range(128 // L)]
            while len(cands) > 1:
                cands = [merge_pair(cands[i], cands[i + 1])
                         for i in range(0, len(cands), 2)]
            vd, vi = cands[0]
            svals = [(vd[lidx], vi[lidx]) for lidx in range(L)]
            while len(svals) > 1:
                nxt = []
                for a in range(0, len(svals), 2):
                    da, ia = svals[a]
                    db, ib = svals[a + 1]
                    t = (db < da) | ((db == da) & (ib < ia))
                    nxt.append((jnp.where(t, db, da), jnp.where(t, ib, ia)))
                svals = nxt
            best_d, best_i = svals[0]
            lane = iota16 == ql
            bd = jnp.where(lane, best_d, bd)
            bi = jnp.where(lane, best_i, bi)
        for r in range(NW):
            dr = dbuf[r, pl.ds(j * L, L)]
            ir = ibuf[r, pl.ds(j * L, L)]
            lt = (dr < bd) | ((dr == bd) & (ir < bi))
            bd = jnp.where(lt, dr, bd)
            bi = jnp.where(lt, ir, bi)
        biv[pl.ds(j * L, L)] = bi

    pltpu.async_copy(mx_hbm.at[biv], pxv, sem).wait()
    pltpu.async_copy(my_hbm.at[biv], pyv, sem).wait()
    pltpu.async_copy(ts_hbm.at[biv], linv, sem).wait()

    pltpu.sync_copy(pxv, px_hbm.at[pl.ds(qbase, QPW)])
    pltpu.sync_copy(pyv, py_hbm.at[pl.ds(qbase, QPW)])
    pltpu.sync_copy(linv, lin_hbm.at[pl.ds(qbase, QPW)])


@jax.jit
def kernel(euclidean_data, maze_points, ts_proj):
    ed = euclidean_data.astype(maze_points.dtype)
    qx = ed[:, 0]
    qy = ed[:, 1]
    mxcol = maze_points[:, 0]
    mycol = maze_points[:, 1]

    tc_d2, tc_i2 = _tc_stage(
        qx[:, None], qy[:, None],
        mxcol[:T_TC].reshape(TKB, 128), mycol[:T_TC].reshape(TKB, 128))
    tc_d = tc_d2.reshape(Q, 128)
    tc_i = tc_i2.reshape(Q, 128)

    dall, iall = _stage1_sc(qx, qy, mxcol[T_TC:], mycol[T_TC:])
    px, py, lin = _stage2(tc_d, tc_i, dall, iall, mxcol, mycol, ts_proj)
    projected = jnp.stack([px, py], axis=-1)
    return projected, lin


# speedup vs baseline: 1.0836x; 1.0077x over previous
import functools

import jax
import jax.numpy as jnp
from jax import lax
from jax.experimental import pallas as pl
from jax.experimental.pallas import tpu as pltpu
from jax.experimental.pallas import tpu_sc as plsc

NC = 2
NS = 16
L = 16
NW = NC * NS

Q = 1024
K = 100000

T_TC = 66560
SC_N = K - T_TC
CHUNK = ((SC_N + NW * L - 1) // (NW * L)) * L
SC_KPAD = CHUNK * NW
KG = T_TC + SC_KPAD

QPW = Q // NW
UNROLL = 16
G = 1
KUNROLL = 4
QPAIR = 2
TKB = T_TC // 128
IMAX = 0x7FFFFFFF

_mesh = functools.partial(
    plsc.VectorSubcoreMesh, core_axis_name="c", subcore_axis_name="s")


def _tc_body(qx_ref, qy_ref, mx_ref, my_ref, outd_ref, outi_ref):
    lane_iota = lax.broadcasted_iota(jnp.int32, (8, 128), 1)
    inf8 = jnp.full((8, 128), jnp.inf, jnp.float32)
    zero8 = jnp.zeros((8, 128), jnp.int32)

    def per_qpair(qp, _):
        qxb = [jnp.broadcast_to(qx_ref[pl.ds((qp * 2 + h) * 8, 8), :],
                                (8, 128)) for h in range(QPAIR)]
        qyb = [jnp.broadcast_to(qy_ref[pl.ds((qp * 2 + h) * 8, 8), :],
                                (8, 128)) for h in range(QPAIR)]

        def inner(t, carry):
            bds, bis = carry
            nbds, nbis = [], []
            for u in range(KUNROLL):
                kb = t * KUNROLL + u
                mxb = jnp.broadcast_to(mx_ref[pl.ds(kb, 1), :], (8, 128))
                myb = jnp.broadcast_to(my_ref[pl.ds(kb, 1), :], (8, 128))
                kbs = jnp.full((8, 128), kb, jnp.int32)
                for h in range(QPAIR):
                    a = u * QPAIR + h
                    dx = qxb[h] - mxb
                    dy = qyb[h] - myb
                    d = dx * dx + dy * dy
                    lt = d < bds[a]
                    nbds.append(jnp.where(lt, d, bds[a]))
                    nbis.append(jnp.where(lt, kbs, bis[a]))
            return tuple(nbds), tuple(nbis)

        NA = KUNROLL * QPAIR
        bds, bis = lax.fori_loop(
            0, TKB // KUNROLL, inner, ((inf8,) * NA, (zero8,) * NA))

        for h in range(QPAIR):
            bd = bds[h]
            bi = bis[h] * 128 + lane_iota
            for u in range(1, KUNROLL):
                a = u * QPAIR + h
                du = bds[a]
                iu = bis[a] * 128 + lane_iota
                lt = (du < bd) | ((du == bd) & (iu < bi))
                bd = jnp.where(lt, du, bd)
                bi = jnp.where(lt, iu, bi)
            outd_ref[qp * 2 + h] = bd
            outi_ref[qp * 2 + h] = bi
        return 0

    lax.fori_loop(0, Q // (8 * QPAIR), per_qpair, 0)


_tc_stage = pl.pallas_call(
    _tc_body,
    out_shape=(
        jax.ShapeDtypeStruct((Q // 8, 8, 128), jnp.float32),
        jax.ShapeDtypeStruct((Q // 8, 8, 128), jnp.int32),
    ),
)


@functools.partial(
    pl.kernel,
    out_type=(
        jax.ShapeDtypeStruct((NW, Q), jnp.float32),
        jax.ShapeDtypeStruct((NW, Q), jnp.int32),
    ),
    mesh=_mesh(),
    scratch_types=[
        pltpu.VMEM((CHUNK,), jnp.float32),
        pltpu.VMEM((CHUNK,), jnp.float32),
        pltpu.VMEM((Q,), jnp.float32),
        pltpu.VMEM((Q,), jnp.float32),
        pltpu.VMEM((Q,), jnp.float32),
        pltpu.VMEM((Q,), jnp.int32),
    ],
)
def _stage1_sc(qx_hbm, qy_hbm, mx_hbm, my_hbm, outd_hbm, outi_hbm,
               mxv, myv, qxv, qyv, bdv, biv):
    c = lax.axis_index("c")
    s = lax.axis_index("s")
    w = s * NC + c
    base = jnp.minimum(w * CHUNK, SC_N - CHUNK)

    pltpu.sync_copy(mx_hbm.at[pl.ds(base, CHUNK)], mxv)
    pltpu.sync_copy(my_hbm.at[pl.ds(base, CHUNK)], myv)
    pltpu.sync_copy(qx_hbm, qxv)
    pltpu.sync_copy(qy_hbm, qyv)

    def per_group(g, _):
        qxb = [qxv[pl.ds((g * G + j) * L, L)] for j in range(G)]
        qyb = [qyv[pl.ds((g * G + j) * L, L)] for j in range(G)]

        def inner(i, carry):
            bd, bi = carry
            k0 = i * UNROLL
            mxvec = mxv[pl.ds(k0, UNROLL)]
            myvec = myv[pl.ds(k0, UNROLL)]
            bd, bi = list(bd), list(bi)
            for u in range(UNROLL):
                k = k0 + u
                mxb = jnp.full((L,), mxvec[u], jnp.float32)
                myb = jnp.full((L,), myvec[u], jnp.float32)
                for j in range(G):
                    dx = qxb[j] - mxb
                    dy = qyb[j] - myb
                    d = dx * dx + dy * dy
                    lt = d < bd[j]
                    bd[j] = jnp.where(lt, d, bd[j])
                    bi[j] = jnp.where(lt, T_TC + base + k, bi[j])
            return tuple(bd), tuple(bi)

        bd0 = (jnp.full((L,), jnp.inf, jnp.float32),) * G
        bi0 = (jnp.zeros((L,), jnp.int32),) * G
        bd, bi = lax.fori_loop(0, CHUNK // UNROLL, inner, (bd0, bi0))
        for j in range(G):
            bdv[pl.ds((g * G + j) * L, L)] = bd[j]
            biv[pl.ds((g * G + j) * L, L)] = bi[j]
        return 0

    lax.fori_loop(0, Q // (L * G), per_group, 0)

    pltpu.sync_copy(bdv, outd_hbm.at[w])
    pltpu.sync_copy(biv, outi_hbm.at[w])


@functools.partial(
    pl.kernel,
    out_type=(
        jax.ShapeDtypeStruct((Q,), jnp.float32),
        jax.ShapeDtypeStruct((Q,), jnp.float32),
        jax.ShapeDtypeStruct((Q,), jnp.float32),
    ),
    mesh=_mesh(),
    scratch_types=[
        pltpu.VMEM((NW, QPW), jnp.float32),
        pltpu.VMEM((NW, QPW), jnp.int32),
        pltpu.VMEM((QPW, 128), jnp.float32),
        pltpu.VMEM((QPW, 128), jnp.int32),
        pltpu.VMEM((QPW,), jnp.int32),
        pltpu.VMEM((QPW,), jnp.float32),
        pltpu.VMEM((QPW,), jnp.float32),
        pltpu.VMEM((QPW,), jnp.float32),
        pltpu.SemaphoreType.DMA,
    ],
)
def _stage2(tcd_hbm, tci_hbm, dall_hbm, iall_hbm, mx_hbm, my_hbm, ts_hbm,
            px_hbm, py_hbm, lin_hbm,
            dbuf, ibuf, tdbuf, tibuf, biv, pxv, pyv, linv, sem):
    c = lax.axis_index("c")
    s = lax.axis_index("s")
    w = s * NC + c
    qbase = w * QPW

    copies = [
        pltpu.async_copy(tcd_hbm.at[pl.ds(qbase, QPW)], tdbuf, sem),
        pltpu.async_copy(tci_hbm.at[pl.ds(qbase, QPW)], tibuf, sem),
    ]
    for r in range(NW):
        copies.append(
            pltpu.async_copy(dall_hbm.at[r, pl.ds(qbase, QPW)], dbuf.at[r], sem))
        copies.append(
            pltpu.async_copy(iall_hbm.at[r, pl.ds(qbase, QPW)], ibuf.at[r], sem))
    for cp in copies:
        cp.wait()

    iota16 = lax.iota(jnp.int32, L)

    def merge_pair(a, b):
        da, ia = a
        db, ib = b
        take_b = (db < da) | ((db == da) & (ib < ia))
        return (jnp.where(take_b, db, da), jnp.where(take_b, ib, ia))

    for j in range(QPW // L):
        bd = jnp.full((L,), jnp.inf, jnp.float32)
        bi = jnp.zeros((L,), jnp.int32)
        for ql in range(L):
            q = j * L + ql
            cands = [(tdbuf[q, pl.ds(cc * L, L)], tibuf[q, pl.ds(cc * L, L)])
                     for cc in range(128 // L)]
            while len(cands) > 1:
                cands = [merge_pair(cands[i], cands[i + 1])
                         for i in range(0, len(cands), 2)]
            vd, vi = cands[0]
            svals = [(vd[lidx], vi[lidx]) for lidx in range(L)]
            while len(svals) > 1:
                nxt = []
                for a in range(0, len(svals), 2):
                    da, ia = svals[a]
                    db, ib = svals[a + 1]
                    t = (db < da) | ((db == da) & (ib < ia))
                    nxt.append((jnp.where(t, db, da), jnp.where(t, ib, ia)))
                svals = nxt
            best_d, best_i = svals[0]
            lane = iota16 == ql
            bd = jnp.where(lane, best_d, bd)
            bi = jnp.where(lane, best_i, bi)
        for r in range(NW):
            dr = dbuf[r, pl.ds(j * L, L)]
            ir = ibuf[r, pl.ds(j * L, L)]
            lt = (dr < bd) | ((dr == bd) & (ir < bi))
            bd = jnp.where(lt, dr, bd)
            bi = jnp.where(lt, ir, bi)
        biv[pl.ds(j * L, L)] = bi

    pltpu.async_copy(mx_hbm.at[biv], pxv, sem).wait()
    pltpu.async_copy(my_hbm.at[biv], pyv, sem).wait()
    pltpu.async_copy(ts_hbm.at[biv], linv, sem).wait()

    pltpu.sync_copy(pxv, px_hbm.at[pl.ds(qbase, QPW)])
    pltpu.sync_copy(pyv, py_hbm.at[pl.ds(qbase, QPW)])
    pltpu.sync_copy(linv, lin_hbm.at[pl.ds(qbase, QPW)])


@jax.jit
def kernel(euclidean_data, maze_points, ts_proj):
    ed = euclidean_data.astype(maze_points.dtype)
    qx = ed[:, 0]
    qy = ed[:, 1]
    mxcol = maze_points[:, 0]
    mycol = maze_points[:, 1]

    tc_d2, tc_i2 = _tc_stage(
        qx[:, None], qy[:, None],
        mxcol[:T_TC].reshape(TKB, 128), mycol[:T_TC].reshape(TKB, 128))
    tc_d = tc_d2.reshape(Q, 128)
    tc_i = tc_i2.reshape(Q, 128)

    dall, iall = _stage1_sc(qx, qy, mxcol[T_TC:], mycol[T_TC:])
    px, py, lin = _stage2(tc_d, tc_i, dall, iall, mxcol, mycol, ts_proj)
    projected = jnp.stack([px, py], axis=-1)
    return projected, lin


# speedup vs baseline: 1.1337x; 1.0463x over previous
import functools

import jax
import jax.numpy as jnp
from jax import lax
from jax.experimental import pallas as pl
from jax.experimental.pallas import tpu as pltpu
from jax.experimental.pallas import tpu_sc as plsc

NC = 2
NS = 16
L = 16
NW = NC * NS

Q = 1024
K = 100000

T_TC = 68608
SC_N = K - T_TC
CHUNK = ((SC_N + NW * L - 1) // (NW * L)) * L
SC_KPAD = CHUNK * NW
KG = T_TC + SC_KPAD

QPW = Q // NW
UNROLL = 16
G = 1
KUNROLL = 4
QPAIR = 2
TKB = T_TC // 128
IMAX = 0x7FFFFFFF

_mesh = functools.partial(
    plsc.VectorSubcoreMesh, core_axis_name="c", subcore_axis_name="s")


def _tc_body(qx_ref, qy_ref, mx_ref, my_ref, outd_ref, outi_ref):
    lane_iota = lax.broadcasted_iota(jnp.int32, (8, 128), 1)
    inf8 = jnp.full((8, 128), jnp.inf, jnp.float32)
    zero8 = jnp.zeros((8, 128), jnp.int32)

    def per_qpair(qp, _):
        qxb = [jnp.broadcast_to(qx_ref[pl.ds((qp * 2 + h) * 8, 8), :],
                                (8, 128)) for h in range(QPAIR)]
        qyb = [jnp.broadcast_to(qy_ref[pl.ds((qp * 2 + h) * 8, 8), :],
                                (8, 128)) for h in range(QPAIR)]

        def inner(t, carry):
            bds, bis = carry
            nbds, nbis = [], []
            for u in range(KUNROLL):
                kb = t * KUNROLL + u
                mxb = jnp.broadcast_to(mx_ref[pl.ds(kb, 1), :], (8, 128))
                myb = jnp.broadcast_to(my_ref[pl.ds(kb, 1), :], (8, 128))
                kbs = jnp.full((8, 128), kb, jnp.int32)
                for h in range(QPAIR):
                    a = u * QPAIR + h
                    dx = qxb[h] - mxb
                    dy = qyb[h] - myb
                    d = dx * dx + dy * dy
                    lt = d < bds[a]
                    nbds.append(jnp.where(lt, d, bds[a]))
                    nbis.append(jnp.where(lt, kbs, bis[a]))
            return tuple(nbds), tuple(nbis)

        NA = KUNROLL * QPAIR
        bds, bis = lax.fori_loop(
            0, TKB // KUNROLL, inner, ((inf8,) * NA, (zero8,) * NA))

        for h in range(QPAIR):
            bd = bds[h]
            bi = bis[h] * 128 + lane_iota
            for u in range(1, KUNROLL):
                a = u * QPAIR + h
                du = bds[a]
                iu = bis[a] * 128 + lane_iota
                lt = (du < bd) | ((du == bd) & (iu < bi))
                bd = jnp.where(lt, du, bd)
                bi = jnp.where(lt, iu, bi)
            outd_ref[qp * 2 + h] = bd
            outi_ref[qp * 2 + h] = bi
        return 0

    lax.fori_loop(0, Q // (8 * QPAIR), per_qpair, 0)


_tc_stage = pl.pallas_call(
    _tc_body,
    out_shape=(
        jax.ShapeDtypeStruct((Q // 8, 8, 128), jnp.float32),
        jax.ShapeDtypeStruct((Q // 8, 8, 128), jnp.int32),
    ),
)


@functools.partial(
    pl.kernel,
    out_type=(
        jax.ShapeDtypeStruct((NW, Q), jnp.float32),
        jax.ShapeDtypeStruct((NW, Q), jnp.int32),
    ),
    mesh=_mesh(),
    scratch_types=[
        pltpu.VMEM((CHUNK,), jnp.float32),
        pltpu.VMEM((CHUNK,), jnp.float32),
        pltpu.VMEM((Q,), jnp.float32),
        pltpu.VMEM((Q,), jnp.float32),
        pltpu.VMEM((Q,), jnp.float32),
        pltpu.VMEM((Q,), jnp.int32),
    ],
)
def _stage1_sc(qx_hbm, qy_hbm, mx_hbm, my_hbm, outd_hbm, outi_hbm,
               mxv, myv, qxv, qyv, bdv, biv):
    c = lax.axis_index("c")
    s = lax.axis_index("s")
    w = s * NC + c
    base = jnp.minimum(w * CHUNK, SC_N - CHUNK)

    pltpu.sync_copy(mx_hbm.at[pl.ds(base, CHUNK)], mxv)
    pltpu.sync_copy(my_hbm.at[pl.ds(base, CHUNK)], myv)
    pltpu.sync_copy(qx_hbm, qxv)
    pltpu.sync_copy(qy_hbm, qyv)

    def per_group(g, _):
        qxb = [qxv[pl.ds((g * G + j) * L, L)] for j in range(G)]
        qyb = [qyv[pl.ds((g * G + j) * L, L)] for j in range(G)]

        def inner(i, carry):
            bd, bi = carry
            k0 = i * UNROLL
            mxvec = mxv[pl.ds(k0, UNROLL)]
            myvec = myv[pl.ds(k0, UNROLL)]
            bd, bi = list(bd), list(bi)
            for u in range(UNROLL):
                k = k0 + u
                mxb = jnp.full((L,), mxvec[u], jnp.float32)
                myb = jnp.full((L,), myvec[u], jnp.float32)
                for j in range(G):
                    dx = qxb[j] - mxb
                    dy = qyb[j] - myb
                    d = dx * dx + dy * dy
                    lt = d < bd[j]
                    bd[j] = jnp.where(lt, d, bd[j])
                    bi[j] = jnp.where(lt, T_TC + base + k, bi[j])
            return tuple(bd), tuple(bi)

        bd0 = (jnp.full((L,), jnp.inf, jnp.float32),) * G
        bi0 = (jnp.zeros((L,), jnp.int32),) * G
        bd, bi = lax.fori_loop(0, CHUNK // UNROLL, inner, (bd0, bi0))
        for j in range(G):
            bdv[pl.ds((g * G + j) * L, L)] = bd[j]
            biv[pl.ds((g * G + j) * L, L)] = bi[j]
        return 0

    lax.fori_loop(0, Q // (L * G), per_group, 0)

    pltpu.sync_copy(bdv, outd_hbm.at[w])
    pltpu.sync_copy(biv, outi_hbm.at[w])


@functools.partial(
    pl.kernel,
    out_type=(
        jax.ShapeDtypeStruct((Q,), jnp.float32),
        jax.ShapeDtypeStruct((Q,), jnp.float32),
        jax.ShapeDtypeStruct((Q,), jnp.float32),
    ),
    mesh=_mesh(),
    scratch_types=[
        pltpu.VMEM((NW, QPW), jnp.float32),
        pltpu.VMEM((NW, QPW), jnp.int32),
        pltpu.VMEM((QPW, 128), jnp.float32),
        pltpu.VMEM((QPW, 128), jnp.int32),
        pltpu.VMEM((QPW,), jnp.int32),
        pltpu.VMEM((QPW,), jnp.float32),
        pltpu.VMEM((QPW,), jnp.float32),
        pltpu.VMEM((QPW,), jnp.float32),
        pltpu.SemaphoreType.DMA,
    ],
)
def _stage2(tcd_hbm, tci_hbm, dall_hbm, iall_hbm, mx_hbm, my_hbm, ts_hbm,
            px_hbm, py_hbm, lin_hbm,
            dbuf, ibuf, tdbuf, tibuf, biv, pxv, pyv, linv, sem):
    c = lax.axis_index("c")
    s = lax.axis_index("s")
    w = s * NC + c
    qbase = w * QPW

    copies = [
        pltpu.async_copy(tcd_hbm.at[pl.ds(qbase, QPW)], tdbuf, sem),
        pltpu.async_copy(tci_hbm.at[pl.ds(qbase, QPW)], tibuf, sem),
    ]
    for r in range(NW):
        copies.append(
            pltpu.async_copy(dall_hbm.at[r, pl.ds(qbase, QPW)], dbuf.at[r], sem))
        copies.append(
            pltpu.async_copy(iall_hbm.at[r, pl.ds(qbase, QPW)], ibuf.at[r], sem))
    for cp in copies:
        cp.wait()

    iota16 = lax.iota(jnp.int32, L)

    def merge_pair(a, b):
        da, ia = a
        db, ib = b
        take_b = (db < da) | ((db == da) & (ib < ia))
        return (jnp.where(take_b, db, da), jnp.where(take_b, ib, ia))

    for j in range(QPW // L):
        bd = jnp.full((L,), jnp.inf, jnp.float32)
        bi = jnp.zeros((L,), jnp.int32)
        for ql in range(L):
            q = j * L + ql
            cands = [(tdbuf[q, pl.ds(cc * L, L)], tibuf[q, pl.ds(cc * L, L)])
                     for cc in range(128 // L)]
            while len(cands) > 1:
                cands = [merge_pair(cands[i], cands[i + 1])
                         for i in range(0, len(cands), 2)]
            vd, vi = cands[0]
            svals = [(vd[lidx], vi[lidx]) for lidx in range(L)]
            while len(svals) > 1:
                nxt = []
                for a in range(0, len(svals), 2):
                    da, ia = svals[a]
                    db, ib = svals[a + 1]
                    t = (db < da) | ((db == da) & (ib < ia))
                    nxt.append((jnp.where(t, db, da), jnp.where(t, ib, ia)))
                svals = nxt
            best_d, best_i = svals[0]
            lane = iota16 == ql
            bd = jnp.where(lane, best_d, bd)
            bi = jnp.where(lane, best_i, bi)
        for r in range(NW):
            dr = dbuf[r, pl.ds(j * L, L)]
            ir = ibuf[r, pl.ds(j * L, L)]
            lt = (dr < bd) | ((dr == bd) & (ir < bi))
            bd = jnp.where(lt, dr, bd)
            bi = jnp.where(lt, ir, bi)
        biv[pl.ds(j * L, L)] = bi

    pltpu.async_copy(mx_hbm.at[biv], pxv, sem).wait()
    pltpu.async_copy(my_hbm.at[biv], pyv, sem).wait()
    pltpu.async_copy(ts_hbm.at[biv], linv, sem).wait()

    pltpu.sync_copy(pxv, px_hbm.at[pl.ds(qbase, QPW)])
    pltpu.sync_copy(pyv, py_hbm.at[pl.ds(qbase, QPW)])
    pltpu.sync_copy(linv, lin_hbm.at[pl.ds(qbase, QPW)])


@jax.jit
def kernel(euclidean_data, maze_points, ts_proj):
    ed = euclidean_data.astype(maze_points.dtype)
    qx = ed[:, 0]
    qy = ed[:, 1]
    mxcol = maze_points[:, 0]
    mycol = maze_points[:, 1]

    tc_d2, tc_i2 = _tc_stage(
        qx[:, None], qy[:, None],
        mxcol[:T_TC].reshape(TKB, 128), mycol[:T_TC].reshape(TKB, 128))
    tc_d = tc_d2.reshape(Q, 128)
    tc_i = tc_i2.reshape(Q, 128)

    dall, iall = _stage1_sc(qx, qy, mxcol[T_TC:], mycol[T_TC:])
    px, py, lin = _stage2(tc_d, tc_i, dall, iall, mxcol, mycol, ts_proj)
    projected = jnp.stack([px, py], axis=-1)
    return projected, lin
